# Initial kernel scaffold; baseline (speedup 1.0000x reference)
import jax
import jax.numpy as jnp
from jax.experimental import pallas as pl


def kernel(h, x, edge_index, params):
    raise NotImplementedError("write your pallas kernel here")


# speedup vs baseline: 1.8819x; 1.8819x over previous
import functools

import jax
import jax.numpy as jnp
from jax import lax
from jax.experimental import pallas as pl
from jax.experimental.pallas import tpu as pltpu
from jax.experimental.pallas import tpu_sc as plsc

NN = 10000
EE = 160000
H = 256
LAT = 64
ODIM = LAT * 2 + 1
OPAD = 256

NC, NS = 2, 16
NW = NC * NS
GK = 128
SB = EE // GK
RPT = 624
RLAST = NN - RPT * (NS - 1)

BN = 1000
BE = 1000

_F32 = jnp.float32
_BF16 = jnp.bfloat16


def _mm(a, w):
    return jnp.dot(a.astype(_BF16), w.astype(_BF16), preferred_element_type=_F32)


def _rb(v):
    return v.astype(_BF16).astype(_F32)


def _mesh():
    return plsc.VectorSubcoreMesh(
        core_axis_name="c", subcore_axis_name="s",
        num_cores=NC, num_subcores=NS)


@functools.cache
def _gather_add_kernel():
    @functools.partial(
        pl.kernel,
        out_type=jax.ShapeDtypeStruct((EE, H), _F32),
        mesh=_mesh(),
        scratch_types=[
            pltpu.VMEM((GK,), jnp.int32),
            pltpu.VMEM((GK,), jnp.int32),
            pltpu.VMEM((GK, H), _F32),
            pltpu.VMEM((GK, H), _F32),
            pltpu.SemaphoreType.DMA,
        ],
    )
    def run(a_hbm, b_hbm, ri_hbm, ci_hbm, out_hbm, ia_v, ib_v, a_v, b_v, sem):
        c = lax.axis_index("c")
        s = lax.axis_index("s")
        w = s * NC + c
        nt = (SB - w + NW - 1) // NW

        def step(t, carry):
            off = (w + t * NW) * GK
            pltpu.sync_copy(ri_hbm.at[pl.ds(off, GK)], ia_v)
            pltpu.sync_copy(ci_hbm.at[pl.ds(off, GK)], ib_v)
            ca = pltpu.async_copy(a_hbm.at[ia_v], a_v, sem)
            cb = pltpu.async_copy(b_hbm.at[ib_v], b_v, sem)
            ca.wait()
            cb.wait()

            def add_row(r, carry2):
                for j in range(H // 16):
                    sl = pl.ds(j * 16, 16)
                    a_v[r, sl] = a_v[r, sl] + b_v[r, sl]
                return carry2

            lax.fori_loop(0, GK, add_row, 0)
            pltpu.sync_copy(a_v, out_hbm.at[pl.ds(off, GK)])
            return carry

        lax.fori_loop(0, nt, step, 0)

    return run


def _sc_gather_add(a, b, ri, ci):
    return _gather_add_kernel()(a, b, ri, ci)


@functools.cache
def _gather_diff_kernel():
    @functools.partial(
        pl.kernel,
        out_type=jax.ShapeDtypeStruct((EE, 16), _F32),
        mesh=_mesh(),
        scratch_types=[
            pltpu.VMEM((GK,), jnp.int32),
            pltpu.VMEM((GK,), jnp.int32),
            pltpu.VMEM((GK, 128), _F32),
            pltpu.VMEM((GK, 128), _F32),
            pltpu.VMEM((GK, 16), _F32),
            pltpu.SemaphoreType.DMA,
        ],
    )
    def run(x_hbm, ri_hbm, ci_hbm, out_hbm, ia_v, ib_v, a_v, b_v, o_v, sem):
        c = lax.axis_index("c")
        s = lax.axis_index("s")
        w = s * NC + c
        nt = (SB - w + NW - 1) // NW

        def step(t, carry):
            off = (w + t * NW) * GK
            pltpu.sync_copy(ri_hbm.at[pl.ds(off, GK)], ia_v)
            pltpu.sync_copy(ci_hbm.at[pl.ds(off, GK)], ib_v)
            ca = pltpu.async_copy(x_hbm.at[ia_v], a_v, sem)
            cb = pltpu.async_copy(x_hbm.at[ib_v], b_v, sem)
            ca.wait()
            cb.wait()

            def sub_row(r, carry2):
                sl = pl.ds(0, 16)
                o_v[r, sl] = a_v[r, sl] - b_v[r, sl]
                return carry2

            lax.fori_loop(0, GK, sub_row, 0)
            pltpu.sync_copy(o_v, out_hbm.at[pl.ds(off, GK)])
            return carry

        lax.fori_loop(0, nt, step, 0)

    return run


def _sc_gather_diff(x16, ri, ci):
    return _gather_diff_kernel()(x16, ri, ci)


@functools.cache
def _scatter_add_kernel():
    hw = H // NC

    @functools.partial(
        pl.kernel,
        out_type=jax.ShapeDtypeStruct((NN, H), _F32),
        mesh=_mesh(),
        scratch_types=[
            pltpu.VMEM((GK,), jnp.int32),
            pltpu.VMEM((GK, hw), _F32),
            pltpu.VMEM_SHARED((NN, hw), _F32),
        ],
    )
    def run(m_hbm, ri_hbm, z_hbm, out_hbm, idx_v, v_v, acc):
        c = lax.axis_index("c")
        s = lax.axis_index("s")
        r0 = s * RPT

        @pl.when(s < NS - 1)
        def _():
            pltpu.sync_copy(z_hbm.at[pl.ds(0, RPT)], acc.at[pl.ds(r0, RPT)])

        @pl.when(s == NS - 1)
        def _():
            pltpu.sync_copy(z_hbm, acc.at[pl.ds(r0, RLAST)])

        plsc.subcore_barrier()
        nt = (SB - s + NS - 1) // NS

        def step(t, carry):
            off = (s + t * NS) * GK
            pltpu.sync_copy(ri_hbm.at[pl.ds(off, GK)], idx_v)
            pltpu.sync_copy(m_hbm.at[pl.ds(off, GK), pl.ds(c * hw, hw)], v_v)
            pltpu.sync_copy(v_v, acc.at[idx_v], add=True)
            return carry

        lax.fori_loop(0, nt, step, 0)
        plsc.subcore_barrier()

        @pl.when(s < NS - 1)
        def _():
            pltpu.sync_copy(acc.at[pl.ds(r0, RPT)],
                            out_hbm.at[pl.ds(r0, RPT), pl.ds(c * hw, hw)])

        @pl.when(s == NS - 1)
        def _():
            pltpu.sync_copy(acc.at[pl.ds(r0, RLAST)],
                            out_hbm.at[pl.ds(r0, RLAST), pl.ds(c * hw, hw)])

    return run


def _sc_scatter_add(m, ri, z):
    return _scatter_add_kernel()(m, ri, z)


@functools.cache
def _scatter_add16_kernel():
    @functools.partial(
        pl.kernel,
        out_type=jax.ShapeDtypeStruct((NC, NN, 128), _F32),
        mesh=_mesh(),
        scratch_types=[
            pltpu.VMEM((GK,), jnp.int32),
            pltpu.VMEM((GK, 128), _F32),
            pltpu.VMEM_SHARED((NN, 128), _F32),
        ],
    )
    def run(t_hbm, ri_hbm, z_hbm, out_hbm, idx_v, v_v, acc):
        c = lax.axis_index("c")
        s = lax.axis_index("s")
        w = s * NC + c
        r0 = s * RPT

        @pl.when(s < NS - 1)
        def _():
            pltpu.sync_copy(z_hbm.at[pl.ds(0, RPT)], acc.at[pl.ds(r0, RPT)])

        @pl.when(s == NS - 1)
        def _():
            pltpu.sync_copy(z_hbm, acc.at[pl.ds(r0, RLAST)])

        plsc.subcore_barrier()
        nt = (SB - w + NW - 1) // NW

        def step(t, carry):
            off = (w + t * NW) * GK
            pltpu.sync_copy(ri_hbm.at[pl.ds(off, GK)], idx_v)
            pltpu.sync_copy(t_hbm.at[pl.ds(off, GK)], v_v)
            pltpu.sync_copy(v_v, acc.at[idx_v], add=True)
            return carry

        lax.fori_loop(0, nt, step, 0)
        plsc.subcore_barrier()

        @pl.when(s < NS - 1)
        def _():
            pltpu.sync_copy(acc.at[pl.ds(r0, RPT)],
                            out_hbm.at[c, pl.ds(r0, RPT)])

        @pl.when(s == NS - 1)
        def _():
            pltpu.sync_copy(acc.at[pl.ds(r0, RLAST)],
                            out_hbm.at[c, pl.ds(r0, RLAST)])

    return run


def _sc_scatter_add16(t16, ri, z):
    return _scatter_add16_kernel()(t16, ri, z)


def _row_spec(b, cdim):
    return pl.BlockSpec((b, cdim), lambda i: (i, 0))


def _full_spec(r, cdim):
    return pl.BlockSpec((r, cdim), lambda i: (0, 0))


def _silu(v):
    return v * jax.nn.sigmoid(v)


def _dual_lin_body(h_ref, wa_ref, ba_ref, wb_ref, oa_ref, ob_ref):
    hb = h_ref[...]
    oa_ref[...] = _mm(hb, wa_ref[...]) + ba_ref[...]
    ob_ref[...] = _mm(hb, wb_ref[...])


def _tc_dual_linear(h, wa, ba, wb):
    return pl.pallas_call(
        _dual_lin_body,
        grid=(NN // BN,),
        in_specs=[_row_spec(BN, H), _full_spec(H, H), _full_spec(1, H),
                  _full_spec(H, H)],
        out_specs=[_row_spec(BN, H), _row_spec(BN, H)],
        out_shape=[jax.ShapeDtypeStruct((NN, H), _F32),
                   jax.ShapeDtypeStruct((NN, H), _F32)],
    )(h, wa, ba, wb)


def _edge_mlp_body(g_ref, d_ref, d0_ref, wc_ref, w2_ref, b2_ref, o_ref):
    g = g_ref[...]
    d = d_ref[...]
    d0 = d0_ref[...]
    rad = jnp.sum(d * d, axis=1, keepdims=True)
    r0 = jnp.sum(d0 * d0, axis=1, keepdims=True)
    wc = wc_ref[...]
    wcb = _rb(wc)
    t = g + _rb(rad) * wcb[0:1, :] + _rb(r0) * wcb[1:2, :]
    t = _silu(t)
    m = _mm(t, w2_ref[...]) + b2_ref[...]
    o_ref[...] = _silu(m)


def _tc_edge_mlp(g, d, d0, wc, w2, b2):
    return pl.pallas_call(
        _edge_mlp_body,
        grid=(EE // BE,),
        in_specs=[_row_spec(BE, H), _row_spec(BE, 16), _row_spec(BE, 16),
                  _full_spec(8, H), _full_spec(H, H), _full_spec(1, H)],
        out_specs=_row_spec(BE, H),
        out_shape=jax.ShapeDtypeStruct((EE, H), _F32),
    )(g, d, d0, wc, w2, b2)


def _node_body(h_ref, a_ref, w1a_ref, w1b_ref, b1_ref, w2_ref, b2_ref, o_ref):
    hb = h_ref[...]
    ab = a_ref[...]
    t = (_mm(hb, w1a_ref[...])
         + _mm(ab, w1b_ref[...])
         + b1_ref[...])
    t = _silu(t)
    o_ref[...] = hb + _mm(t, w2_ref[...]) + b2_ref[...]


def _tc_node_mlp(h, agg, w1a, w1b, b1, w2, b2):
    return pl.pallas_call(
        _node_body,
        grid=(NN // BN,),
        in_specs=[_row_spec(BN, H), _row_spec(BN, H), _full_spec(H, H),
                  _full_spec(H, H), _full_spec(1, H), _full_spec(H, H),
                  _full_spec(1, H)],
        out_specs=_row_spec(BN, H),
        out_shape=jax.ShapeDtypeStruct((NN, H), _F32),
    )(h, agg, w1a, w1b, b1, w2, b2)


def _coord_body(g_ref, d_ref, d0_ref, wc_ref, w2_ref, b2_ref, w3_ref, o_ref):
    g = g_ref[...]
    d = d_ref[...]
    d0 = d0_ref[...]
    rad = jnp.sum(d * d, axis=1, keepdims=True)
    r0 = jnp.sum(d0 * d0, axis=1, keepdims=True)
    wc = wc_ref[...]
    wcb = _rb(wc)
    t = g + _rb(rad) * wcb[0:1, :] + _rb(r0) * wcb[1:2, :]
    t = _silu(t)
    m = _silu(_mm(t, w2_ref[...]) + b2_ref[...])
    sval = jnp.sum(_rb(m) * _rb(w3_ref[0:1, :]), axis=1, keepdims=True)
    inv = 1.0 / jnp.sqrt(rad + 1e-8)
    t16 = d * (sval * inv)
    o_ref[...] = jnp.concatenate(
        [t16, jnp.zeros((t16.shape[0], 112), _F32)], axis=1)


def _tc_coord_edge(g, d, d0, wc, w2, b2, w3):
    return pl.pallas_call(
        _coord_body,
        grid=(EE // BE,),
        in_specs=[_row_spec(BE, H), _row_spec(BE, 16), _row_spec(BE, 16),
                  _full_spec(8, H), _full_spec(H, H), _full_spec(1, H),
                  _full_spec(8, H)],
        out_specs=_row_spec(BE, 128),
        out_shape=jax.ShapeDtypeStruct((EE, 128), _F32),
    )(g, d, d0, wc, w2, b2, w3)


def _xupd_body(x_ref, a_ref, o_ref):
    o_ref[...] = x_ref[...] + a_ref[0, :, :16] + a_ref[1, :, :16]


def _tc_x_update(x16, aggt):
    return pl.pallas_call(
        _xupd_body,
        grid=(NN // BN,),
        in_specs=[_row_spec(BN, 16),
                  pl.BlockSpec((NC, BN, 128), lambda i: (0, i, 0))],
        out_specs=_row_spec(BN, 16),
        out_shape=jax.ShapeDtypeStruct((NN, 16), _F32),
    )(x16, aggt)


def _emb_body(h_ref, w_ref, b_ref, o_ref):
    o_ref[...] = _mm(h_ref[...], w_ref[...]) + b_ref[...]


def _tc_linear(h, w, b):
    return pl.pallas_call(
        _emb_body,
        grid=(NN // BN,),
        in_specs=[_row_spec(BN, H), _full_spec(H, H), _full_spec(1, H)],
        out_specs=_row_spec(BN, H),
        out_shape=jax.ShapeDtypeStruct((NN, H), _F32),
    )(h, w, b)


def _head_body(h_ref, wo_ref, bo_ref, w1_ref, b1_ref, w2_ref, b2_ref, o_ref):
    hb = h_ref[...]
    o = _mm(hb, wo_ref[...]) + bo_ref[...]
    f = _silu(_mm(o, w1_ref[...]) + b1_ref[...])
    o_ref[...] = _mm(f, w2_ref[...]) + b2_ref[...]


def _tc_head(h, wo, bo, w1, b1, w2p, b2p):
    return pl.pallas_call(
        _head_body,
        grid=(NN // BN,),
        in_specs=[_row_spec(BN, H), _full_spec(H, H), _full_spec(1, H),
                  _full_spec(H, H), _full_spec(1, H), _full_spec(H, OPAD),
                  _full_spec(1, OPAD)],
        out_specs=_row_spec(BN, OPAD),
        out_shape=jax.ShapeDtypeStruct((NN, OPAD), _F32),
    )(h, wo, bo, w1, b1, w2p, b2p)


def _pad_wc(wc2):
    return jnp.zeros((8, H), _F32).at[:2].set(wc2)


def kernel(h, x, edge_index, params):
    row = edge_index[0].astype(jnp.int32)
    col = edge_index[1].astype(jnp.int32)
    x16 = jnp.zeros((NN, 16), _F32).at[:, :3].set(x)

    def _xpad(x16v):
        return jnp.zeros((NN, 128), _F32).at[:, :16].set(x16v)

    z128 = jnp.zeros((RLAST, H // NC), _F32)

    h = _tc_linear(h, params['emb_W'], params['emb_b'].reshape(1, H))

    d0 = _sc_gather_diff(_xpad(x16), row, col)
    d = d0
    first = True
    for bp in params['blocks']:
        if not first:
            d = _sc_gather_diff(_xpad(x16), row, col)
        first = False
        for gp in bp['gcls']:
            wa = gp['e1_W'][:H]
            wb = gp['e1_W'][H:2 * H]
            wc = _pad_wc(gp['e1_W'][2 * H:])
            ha, hb = _tc_dual_linear(h, wa, gp['e1_b'].reshape(1, H), wb)
            g = _sc_gather_add(ha, hb, row, col)
            mij = _tc_edge_mlp(g, d, d0, wc, gp['e2_W'],
                               gp['e2_b'].reshape(1, H))
            agg = _sc_scatter_add(mij, row, z128)
            h = _tc_node_mlp(h, agg,
                             gp['n1_W'][:H], gp['n1_W'][H:],
                             gp['n1_b'].reshape(1, H),
                             gp['n2_W'], gp['n2_b'].reshape(1, H))
        cp = bp['coord']
        wa = cp['c1_W'][:H]
        wb = cp['c1_W'][H:2 * H]
        wc = _pad_wc(cp['c1_W'][2 * H:])
        ha, hb = _tc_dual_linear(h, wa, cp['c1_b'].reshape(1, H), wb)
        gc = _sc_gather_add(ha, hb, row, col)
        w3 = jnp.zeros((8, H), _F32).at[0].set(cp['c3_W'][:, 0])
        t16 = _tc_coord_edge(gc, d, d0, wc, cp['c2_W'],
                             cp['c2_b'].reshape(1, H), w3)
        aggt = _sc_scatter_add16(t16, row, z128)
        x16 = _tc_x_update(x16, aggt)

    w2p = jnp.zeros((H, OPAD), _F32).at[:, :ODIM].set(params['f2_W'])
    b2p = jnp.zeros((1, OPAD), _F32).at[0, :ODIM].set(params['f2_b'])
    y = _tc_head(h, params['out_W'], params['out_b'].reshape(1, H),
                 params['f1_W'], params['f1_b'].reshape(1, H), w2p, b2p)
    return y[:, :ODIM]


# speedup vs baseline: 2.2160x; 1.1775x over previous
import functools

import jax
import jax.numpy as jnp
from jax import lax
from jax.experimental import pallas as pl
from jax.experimental.pallas import tpu as pltpu
from jax.experimental.pallas import tpu_sc as plsc

NN = 10000
EE = 160000
H = 256
LAT = 64
ODIM = LAT * 2 + 1
OPAD = 256

NC, NS = 2, 16
NW = NC * NS
GK = 128
SB = EE // GK
RPT = 624
RLAST = NN - RPT * (NS - 1)

BN = 1000
BE = 1000

_F32 = jnp.float32
_BF16 = jnp.bfloat16


def _mm(a, w):
    return jnp.dot(a.astype(_BF16), w.astype(_BF16), preferred_element_type=_F32)


def _rb(v):
    return v.astype(_BF16).astype(_F32)


def _mesh():
    return plsc.VectorSubcoreMesh(
        core_axis_name="c", subcore_axis_name="s",
        num_cores=NC, num_subcores=NS)


@functools.cache
def _gather_add_kernel():
    @functools.partial(
        pl.kernel,
        out_type=(jax.ShapeDtypeStruct((EE, 128), jnp.int32),
                  jax.ShapeDtypeStruct((EE, 128), jnp.int32)),
        mesh=_mesh(),
        scratch_types=[
            pltpu.VMEM((GK,), jnp.int32),
            pltpu.VMEM((GK,), jnp.int32),
            pltpu.VMEM((GK, 128), jnp.int32),
            pltpu.VMEM((GK, 128), jnp.int32),
            pltpu.SemaphoreType.DMA,
        ],
    )
    def run(a_hbm, b_hbm, ri_hbm, ci_hbm, oa_hbm, ob_hbm,
            ia_v, ib_v, a_v, b_v, sem):
        c = lax.axis_index("c")
        s = lax.axis_index("s")
        w = s * NC + c
        nt = (SB - w + NW - 1) // NW

        def step(t, carry):
            off = (w + t * NW) * GK
            pltpu.sync_copy(ri_hbm.at[pl.ds(off, GK)], ia_v)
            pltpu.sync_copy(ci_hbm.at[pl.ds(off, GK)], ib_v)
            ca = pltpu.async_copy(a_hbm.at[ia_v], a_v, sem)
            cb = pltpu.async_copy(b_hbm.at[ib_v], b_v, sem)
            ca.wait()
            cb.wait()
            pltpu.sync_copy(a_v, oa_hbm.at[pl.ds(off, GK)])
            pltpu.sync_copy(b_v, ob_hbm.at[pl.ds(off, GK)])
            return carry

        lax.fori_loop(0, nt, step, 0)

    return run


def _sc_gather_add(a, b, ri, ci):
    return _gather_add_kernel()(a, b, ri, ci)


@functools.cache
def _gather_diff_kernel():
    @functools.partial(
        pl.kernel,
        out_type=jax.ShapeDtypeStruct((EE, 16), _F32),
        mesh=_mesh(),
        scratch_types=[
            pltpu.VMEM((GK,), jnp.int32),
            pltpu.VMEM((GK,), jnp.int32),
            pltpu.VMEM((GK, 128), _F32),
            pltpu.VMEM((GK, 128), _F32),
            pltpu.VMEM((GK, 16), _F32),
            pltpu.SemaphoreType.DMA,
        ],
    )
    def run(x_hbm, ri_hbm, ci_hbm, out_hbm, ia_v, ib_v, a_v, b_v, o_v, sem):
        c = lax.axis_index("c")
        s = lax.axis_index("s")
        w = s * NC + c
        nt = (SB - w + NW - 1) // NW

        def step(t, carry):
            off = (w + t * NW) * GK
            pltpu.sync_copy(ri_hbm.at[pl.ds(off, GK)], ia_v)
            pltpu.sync_copy(ci_hbm.at[pl.ds(off, GK)], ib_v)
            ca = pltpu.async_copy(x_hbm.at[ia_v], a_v, sem)
            cb = pltpu.async_copy(x_hbm.at[ib_v], b_v, sem)
            ca.wait()
            cb.wait()

            def sub_row(r, carry2):
                sl = pl.ds(0, 16)
                o_v[r, sl] = a_v[r, sl] - b_v[r, sl]
                return carry2

            lax.fori_loop(0, GK, sub_row, 0)
            pltpu.sync_copy(o_v, out_hbm.at[pl.ds(off, GK)])
            return carry

        lax.fori_loop(0, nt, step, 0)

    return run


def _sc_gather_diff(x16, ri, ci):
    return _gather_diff_kernel()(x16, ri, ci)


@functools.cache
def _scatter_add_kernel():
    hw = H // NC

    @functools.partial(
        pl.kernel,
        out_type=jax.ShapeDtypeStruct((NN, H), _F32),
        mesh=_mesh(),
        scratch_types=[
            pltpu.VMEM((GK,), jnp.int32),
            pltpu.VMEM((GK, hw), _F32),
            pltpu.VMEM_SHARED((NN, hw), _F32),
        ],
    )
    def run(m_hbm, ri_hbm, z_hbm, out_hbm, idx_v, v_v, acc):
        c = lax.axis_index("c")
        s = lax.axis_index("s")
        r0 = s * RPT

        @pl.when(s < NS - 1)
        def _():
            pltpu.sync_copy(z_hbm.at[pl.ds(0, RPT)], acc.at[pl.ds(r0, RPT)])

        @pl.when(s == NS - 1)
        def _():
            pltpu.sync_copy(z_hbm, acc.at[pl.ds(r0, RLAST)])

        plsc.subcore_barrier()
        nt = (SB - s + NS - 1) // NS

        def step(t, carry):
            off = (s + t * NS) * GK
            pltpu.sync_copy(ri_hbm.at[pl.ds(off, GK)], idx_v)
            pltpu.sync_copy(m_hbm.at[pl.ds(off, GK), pl.ds(c * hw, hw)], v_v)
            pltpu.sync_copy(v_v, acc.at[idx_v], add=True)
            return carry

        lax.fori_loop(0, nt, step, 0)
        plsc.subcore_barrier()

        @pl.when(s < NS - 1)
        def _():
            pltpu.sync_copy(acc.at[pl.ds(r0, RPT)],
                            out_hbm.at[pl.ds(r0, RPT), pl.ds(c * hw, hw)])

        @pl.when(s == NS - 1)
        def _():
            pltpu.sync_copy(acc.at[pl.ds(r0, RLAST)],
                            out_hbm.at[pl.ds(r0, RLAST), pl.ds(c * hw, hw)])

    return run


def _sc_scatter_add(m, ri, z):
    return _scatter_add_kernel()(m, ri, z)


@functools.cache
def _scatter_add16_kernel():
    @functools.partial(
        pl.kernel,
        out_type=jax.ShapeDtypeStruct((NC, NN, 128), _F32),
        mesh=_mesh(),
        scratch_types=[
            pltpu.VMEM((GK,), jnp.int32),
            pltpu.VMEM((GK, 128), _F32),
            pltpu.VMEM_SHARED((NN, 128), _F32),
        ],
    )
    def run(t_hbm, ri_hbm, z_hbm, out_hbm, idx_v, v_v, acc):
        c = lax.axis_index("c")
        s = lax.axis_index("s")
        w = s * NC + c
        r0 = s * RPT

        @pl.when(s < NS - 1)
        def _():
            pltpu.sync_copy(z_hbm.at[pl.ds(0, RPT)], acc.at[pl.ds(r0, RPT)])

        @pl.when(s == NS - 1)
        def _():
            pltpu.sync_copy(z_hbm, acc.at[pl.ds(r0, RLAST)])

        plsc.subcore_barrier()
        nt = (SB - w + NW - 1) // NW

        def step(t, carry):
            off = (w + t * NW) * GK
            pltpu.sync_copy(ri_hbm.at[pl.ds(off, GK)], idx_v)
            pltpu.sync_copy(t_hbm.at[pl.ds(off, GK)], v_v)
            pltpu.sync_copy(v_v, acc.at[idx_v], add=True)
            return carry

        lax.fori_loop(0, nt, step, 0)
        plsc.subcore_barrier()

        @pl.when(s < NS - 1)
        def _():
            pltpu.sync_copy(acc.at[pl.ds(r0, RPT)],
                            out_hbm.at[c, pl.ds(r0, RPT)])

        @pl.when(s == NS - 1)
        def _():
            pltpu.sync_copy(acc.at[pl.ds(r0, RLAST)],
                            out_hbm.at[c, pl.ds(r0, RLAST)])

    return run


def _sc_scatter_add16(t16, ri, z):
    return _scatter_add16_kernel()(t16, ri, z)


def _row_spec(b, cdim):
    return pl.BlockSpec((b, cdim), lambda i: (i, 0))


def _full_spec(r, cdim):
    return pl.BlockSpec((r, cdim), lambda i: (0, 0))


def _silu(v):
    return v * jax.nn.sigmoid(v)


def _pack(v):
    lo = jax.lax.bitcast_convert_type(v[:, :128].astype(_BF16), jnp.uint16)
    hi = jax.lax.bitcast_convert_type(v[:, 128:].astype(_BF16), jnp.uint16)
    word = hi.astype(jnp.uint32) << 16 | lo.astype(jnp.uint32)
    return jax.lax.bitcast_convert_type(word, jnp.int32)


def _unpack(w):
    word = jax.lax.bitcast_convert_type(w, jnp.uint32)
    lo = jax.lax.bitcast_convert_type((word & 0xFFFF).astype(jnp.uint16), _BF16)
    hi = jax.lax.bitcast_convert_type((word >> 16).astype(jnp.uint16), _BF16)
    return lo.astype(_F32), hi.astype(_F32)


def _dual_lin_body(h_ref, wa_ref, ba_ref, wb_ref, oa_ref, ob_ref):
    hb = h_ref[...]
    va = _mm(hb, wa_ref[...]) + ba_ref[...]
    vb = _mm(hb, wb_ref[...])
    oa_ref[...] = _pack(va)
    ob_ref[...] = _pack(vb)


def _tc_dual_linear(h, wa, ba, wb):
    return pl.pallas_call(
        _dual_lin_body,
        grid=(NN // BN,),
        in_specs=[_row_spec(BN, H), _full_spec(H, H), _full_spec(1, H),
                  _full_spec(H, H)],
        out_specs=[_row_spec(BN, 128), _row_spec(BN, 128)],
        out_shape=[jax.ShapeDtypeStruct((NN, 128), jnp.int32),
                   jax.ShapeDtypeStruct((NN, 128), jnp.int32)],
    )(h, wa, ba, wb)


def _edge_mlp_body(ga_ref, gb_ref, d_ref, d0_ref, wc_ref, w2_ref, b2_ref,
                   o_ref):
    ga0, ga1 = _unpack(ga_ref[...])
    gb0, gb1 = _unpack(gb_ref[...])
    g0 = ga0 + gb0
    g1 = ga1 + gb1
    d = d_ref[...]
    d0 = d0_ref[...]
    rad = jnp.sum(d * d, axis=1, keepdims=True)
    r0 = jnp.sum(d0 * d0, axis=1, keepdims=True)
    wcb = _rb(wc_ref[...])
    radb = _rb(rad)
    r0b = _rb(r0)
    t0 = _silu(g0 + radb * wcb[0:1, :128] + r0b * wcb[1:2, :128])
    t1 = _silu(g1 + radb * wcb[0:1, 128:] + r0b * wcb[1:2, 128:])
    m = (_mm(t0, w2_ref[0:128, :]) + _mm(t1, w2_ref[128:, :])
         + b2_ref[...])
    o_ref[...] = _silu(m)


def _tc_edge_mlp(ga, gb, d, d0, wc, w2, b2):
    return pl.pallas_call(
        _edge_mlp_body,
        grid=(EE // BE,),
        in_specs=[_row_spec(BE, 128), _row_spec(BE, 128),
                  _row_spec(BE, 16), _row_spec(BE, 16),
                  _full_spec(8, H), _full_spec(H, H), _full_spec(1, H)],
        out_specs=_row_spec(BE, H),
        out_shape=jax.ShapeDtypeStruct((EE, H), _F32),
    )(ga, gb, d, d0, wc, w2, b2)


def _node_body(h_ref, a_ref, w1a_ref, w1b_ref, b1_ref, w2_ref, b2_ref, o_ref):
    hb = h_ref[...]
    ab = a_ref[...]
    t = (_mm(hb, w1a_ref[...])
         + _mm(ab, w1b_ref[...])
         + b1_ref[...])
    t = _silu(t)
    o_ref[...] = hb + _mm(t, w2_ref[...]) + b2_ref[...]


def _tc_node_mlp(h, agg, w1a, w1b, b1, w2, b2):
    return pl.pallas_call(
        _node_body,
        grid=(NN // BN,),
        in_specs=[_row_spec(BN, H), _row_spec(BN, H), _full_spec(H, H),
                  _full_spec(H, H), _full_spec(1, H), _full_spec(H, H),
                  _full_spec(1, H)],
        out_specs=_row_spec(BN, H),
        out_shape=jax.ShapeDtypeStruct((NN, H), _F32),
    )(h, agg, w1a, w1b, b1, w2, b2)


def _coord_body(ga_ref, gb_ref, d_ref, d0_ref, wc_ref, w2_ref, b2_ref,
                w3_ref, o_ref):
    ga0, ga1 = _unpack(ga_ref[...])
    gb0, gb1 = _unpack(gb_ref[...])
    g0 = ga0 + gb0
    g1 = ga1 + gb1
    d = d_ref[...]
    d0 = d0_ref[...]
    rad = jnp.sum(d * d, axis=1, keepdims=True)
    r0 = jnp.sum(d0 * d0, axis=1, keepdims=True)
    wcb = _rb(wc_ref[...])
    radb = _rb(rad)
    r0b = _rb(r0)
    t0 = _silu(g0 + radb * wcb[0:1, :128] + r0b * wcb[1:2, :128])
    t1 = _silu(g1 + radb * wcb[0:1, 128:] + r0b * wcb[1:2, 128:])
    m = _silu(_mm(t0, w2_ref[0:128, :]) + _mm(t1, w2_ref[128:, :])
              + b2_ref[...])
    sval = jnp.sum(_rb(m) * _rb(w3_ref[0:1, :]), axis=1, keepdims=True)
    inv = 1.0 / jnp.sqrt(rad + 1e-8)
    t16 = d * (sval * inv)
    o_ref[...] = jnp.concatenate(
        [t16, jnp.zeros((t16.shape[0], 112), _F32)], axis=1)


def _tc_coord_edge(ga, gb, d, d0, wc, w2, b2, w3):
    return pl.pallas_call(
        _coord_body,
        grid=(EE // BE,),
        in_specs=[_row_spec(BE, 128), _row_spec(BE, 128),
                  _row_spec(BE, 16), _row_spec(BE, 16),
                  _full_spec(8, H), _full_spec(H, H), _full_spec(1, H),
                  _full_spec(8, H)],
        out_specs=_row_spec(BE, 128),
        out_shape=jax.ShapeDtypeStruct((EE, 128), _F32),
    )(ga, gb, d, d0, wc, w2, b2, w3)


def _xupd_body(x_ref, a_ref, o_ref):
    o_ref[...] = x_ref[...] + a_ref[0, :, :16] + a_ref[1, :, :16]


def _tc_x_update(x16, aggt):
    return pl.pallas_call(
        _xupd_body,
        grid=(NN // BN,),
        in_specs=[_row_spec(BN, 16),
                  pl.BlockSpec((NC, BN, 128), lambda i: (0, i, 0))],
        out_specs=_row_spec(BN, 16),
        out_shape=jax.ShapeDtypeStruct((NN, 16), _F32),
    )(x16, aggt)


def _emb_body(h_ref, w_ref, b_ref, o_ref):
    o_ref[...] = _mm(h_ref[...], w_ref[...]) + b_ref[...]


def _tc_linear(h, w, b):
    return pl.pallas_call(
        _emb_body,
        grid=(NN // BN,),
        in_specs=[_row_spec(BN, H), _full_spec(H, H), _full_spec(1, H)],
        out_specs=_row_spec(BN, H),
        out_shape=jax.ShapeDtypeStruct((NN, H), _F32),
    )(h, w, b)


def _head_body(h_ref, wo_ref, bo_ref, w1_ref, b1_ref, w2_ref, b2_ref, o_ref):
    hb = h_ref[...]
    o = _mm(hb, wo_ref[...]) + bo_ref[...]
    f = _silu(_mm(o, w1_ref[...]) + b1_ref[...])
    o_ref[...] = _mm(f, w2_ref[...]) + b2_ref[...]


def _tc_head(h, wo, bo, w1, b1, w2p, b2p):
    return pl.pallas_call(
        _head_body,
        grid=(NN // BN,),
        in_specs=[_row_spec(BN, H), _full_spec(H, H), _full_spec(1, H),
                  _full_spec(H, H), _full_spec(1, H), _full_spec(H, OPAD),
                  _full_spec(1, OPAD)],
        out_specs=_row_spec(BN, OPAD),
        out_shape=jax.ShapeDtypeStruct((NN, OPAD), _F32),
    )(h, wo, bo, w1, b1, w2p, b2p)


def _pad_wc(wc2):
    return jnp.zeros((8, H), _F32).at[:2].set(wc2)


def kernel(h, x, edge_index, params):
    row = edge_index[0].astype(jnp.int32)
    col = edge_index[1].astype(jnp.int32)
    x16 = jnp.zeros((NN, 16), _F32).at[:, :3].set(x)

    def _xpad(x16v):
        return jnp.zeros((NN, 128), _F32).at[:, :16].set(x16v)

    z128 = jnp.zeros((RLAST, H // NC), _F32)

    h = _tc_linear(h, params['emb_W'], params['emb_b'].reshape(1, H))

    d0 = _sc_gather_diff(_xpad(x16), row, col)
    d = d0
    first = True
    for bp in params['blocks']:
        if not first:
            d = _sc_gather_diff(_xpad(x16), row, col)
        first = False
        for gp in bp['gcls']:
            wa = gp['e1_W'][:H]
            wb = gp['e1_W'][H:2 * H]
            wc = _pad_wc(gp['e1_W'][2 * H:])
            ha, hb = _tc_dual_linear(h, wa, gp['e1_b'].reshape(1, H), wb)
            ga, gb = _sc_gather_add(ha, hb, row, col)
            mij = _tc_edge_mlp(ga, gb, d, d0, wc, gp['e2_W'],
                               gp['e2_b'].reshape(1, H))
            agg = _sc_scatter_add(mij, row, z128)
            h = _tc_node_mlp(h, agg,
                             gp['n1_W'][:H], gp['n1_W'][H:],
                             gp['n1_b'].reshape(1, H),
                             gp['n2_W'], gp['n2_b'].reshape(1, H))
        cp = bp['coord']
        wa = cp['c1_W'][:H]
        wb = cp['c1_W'][H:2 * H]
        wc = _pad_wc(cp['c1_W'][2 * H:])
        ha, hb = _tc_dual_linear(h, wa, cp['c1_b'].reshape(1, H), wb)
        ga, gb = _sc_gather_add(ha, hb, row, col)
        w3 = jnp.zeros((8, H), _F32).at[0].set(cp['c3_W'][:, 0])
        t16 = _tc_coord_edge(ga, gb, d, d0, wc, cp['c2_W'],
                             cp['c2_b'].reshape(1, H), w3)
        aggt = _sc_scatter_add16(t16, row, z128)
        x16 = _tc_x_update(x16, aggt)

    w2p = jnp.zeros((H, OPAD), _F32).at[:, :ODIM].set(params['f2_W'])
    b2p = jnp.zeros((1, OPAD), _F32).at[0, :ODIM].set(params['f2_b'])
    y = _tc_head(h, params['out_W'], params['out_b'].reshape(1, H),
                 params['f1_W'], params['f1_b'].reshape(1, H), w2p, b2p)
    return y[:, :ODIM]


# speedup vs baseline: 2.5806x; 1.1646x over previous
import functools

import jax
import jax.numpy as jnp
from jax import lax
from jax.experimental import pallas as pl
from jax.experimental.pallas import tpu as pltpu
from jax.experimental.pallas import tpu_sc as plsc

NN = 10000
EE = 160000
H = 256
LAT = 64
ODIM = LAT * 2 + 1
OPAD = 256

NC, NS = 2, 16
NW = NC * NS
GK = 128
SB = EE // GK
RPT = 624
RLAST = NN - RPT * (NS - 1)

BN = 1000
BE = 1000

_F32 = jnp.float32
_BF16 = jnp.bfloat16


def _mm(a, w):
    return jnp.dot(a.astype(_BF16), w.astype(_BF16), preferred_element_type=_F32)


def _rb(v):
    return v.astype(_BF16).astype(_F32)


def _mesh():
    return plsc.VectorSubcoreMesh(
        core_axis_name="c", subcore_axis_name="s",
        num_cores=NC, num_subcores=NS)


@functools.cache
def _gather_add_kernel(ne):
    sb = ne // GK

    @functools.partial(
        pl.kernel,
        out_type=(jax.ShapeDtypeStruct((ne, 128), jnp.int32),
                  jax.ShapeDtypeStruct((ne, 128), jnp.int32)),
        mesh=_mesh(),
        scratch_types=[
            pltpu.VMEM((GK,), jnp.int32),
            pltpu.VMEM((GK,), jnp.int32),
            pltpu.VMEM((GK, 128), jnp.int32),
            pltpu.VMEM((GK, 128), jnp.int32),
            pltpu.SemaphoreType.DMA,
        ],
    )
    def run(a_hbm, b_hbm, ri_hbm, ci_hbm, oa_hbm, ob_hbm,
            ia_v, ib_v, a_v, b_v, sem):
        c = lax.axis_index("c")
        s = lax.axis_index("s")
        w = s * NC + c
        nt = (sb - w + NW - 1) // NW

        def step(t, carry):
            off = (w + t * NW) * GK
            pltpu.sync_copy(ri_hbm.at[pl.ds(off, GK)], ia_v)
            pltpu.sync_copy(ci_hbm.at[pl.ds(off, GK)], ib_v)
            ca = pltpu.async_copy(a_hbm.at[ia_v], a_v, sem)
            cb = pltpu.async_copy(b_hbm.at[ib_v], b_v, sem)
            ca.wait()
            cb.wait()
            pltpu.sync_copy(a_v, oa_hbm.at[pl.ds(off, GK)])
            pltpu.sync_copy(b_v, ob_hbm.at[pl.ds(off, GK)])
            return carry

        lax.fori_loop(0, nt, step, 0)

    return run


def _sc_gather_add(a, b, ri, ci):
    return _gather_add_kernel(ri.shape[0])(a, b, ri, ci)


@functools.cache
def _gather_diff_kernel():
    @functools.partial(
        pl.kernel,
        out_type=jax.ShapeDtypeStruct((EE, 16), _F32),
        mesh=_mesh(),
        scratch_types=[
            pltpu.VMEM((GK,), jnp.int32),
            pltpu.VMEM((GK,), jnp.int32),
            pltpu.VMEM((GK, 128), _F32),
            pltpu.VMEM((GK, 128), _F32),
            pltpu.VMEM((GK, 16), _F32),
            pltpu.SemaphoreType.DMA,
        ],
    )
    def run(x_hbm, ri_hbm, ci_hbm, out_hbm, ia_v, ib_v, a_v, b_v, o_v, sem):
        c = lax.axis_index("c")
        s = lax.axis_index("s")
        w = s * NC + c
        nt = (SB - w + NW - 1) // NW

        def step(t, carry):
            off = (w + t * NW) * GK
            pltpu.sync_copy(ri_hbm.at[pl.ds(off, GK)], ia_v)
            pltpu.sync_copy(ci_hbm.at[pl.ds(off, GK)], ib_v)
            ca = pltpu.async_copy(x_hbm.at[ia_v], a_v, sem)
            cb = pltpu.async_copy(x_hbm.at[ib_v], b_v, sem)
            ca.wait()
            cb.wait()

            def sub_row(r, carry2):
                sl = pl.ds(0, 16)
                o_v[r, sl] = a_v[r, sl] - b_v[r, sl]
                return carry2

            lax.fori_loop(0, GK, sub_row, 0)
            pltpu.sync_copy(o_v, out_hbm.at[pl.ds(off, GK)])
            return carry

        lax.fori_loop(0, nt, step, 0)

    return run


def _sc_gather_diff(x16, ri, ci):
    return _gather_diff_kernel()(x16, ri, ci)


@functools.cache
def _scatter_add_kernel(ne):
    hw = H // NC
    sb = ne // GK

    @functools.partial(
        pl.kernel,
        out_type=jax.ShapeDtypeStruct((NN, H), _F32),
        mesh=_mesh(),
        scratch_types=[
            pltpu.VMEM((GK,), jnp.int32),
            pltpu.VMEM((GK, hw), _F32),
            pltpu.VMEM_SHARED((NN, hw), _F32),
        ],
    )
    def run(m_hbm, ri_hbm, z_hbm, out_hbm, idx_v, v_v, acc):
        c = lax.axis_index("c")
        s = lax.axis_index("s")
        r0 = s * RPT

        @pl.when(s < NS - 1)
        def _():
            pltpu.sync_copy(z_hbm.at[pl.ds(0, RPT)], acc.at[pl.ds(r0, RPT)])

        @pl.when(s == NS - 1)
        def _():
            pltpu.sync_copy(z_hbm, acc.at[pl.ds(r0, RLAST)])

        plsc.subcore_barrier()
        nt = (sb - s + NS - 1) // NS

        def step(t, carry):
            off = (s + t * NS) * GK
            pltpu.sync_copy(ri_hbm.at[pl.ds(off, GK)], idx_v)
            pltpu.sync_copy(m_hbm.at[pl.ds(off, GK), pl.ds(c * hw, hw)], v_v)
            pltpu.sync_copy(v_v, acc.at[idx_v], add=True)
            return carry

        lax.fori_loop(0, nt, step, 0)
        plsc.subcore_barrier()

        @pl.when(s < NS - 1)
        def _():
            pltpu.sync_copy(acc.at[pl.ds(r0, RPT)],
                            out_hbm.at[pl.ds(r0, RPT), pl.ds(c * hw, hw)])

        @pl.when(s == NS - 1)
        def _():
            pltpu.sync_copy(acc.at[pl.ds(r0, RLAST)],
                            out_hbm.at[pl.ds(r0, RLAST), pl.ds(c * hw, hw)])

    return run


def _sc_scatter_add(m, ri, z):
    return _scatter_add_kernel(ri.shape[0])(m, ri, z)


@functools.cache
def _scatter_add16_kernel():
    @functools.partial(
        pl.kernel,
        out_type=jax.ShapeDtypeStruct((NC, NN, 128), _F32),
        mesh=_mesh(),
        scratch_types=[
            pltpu.VMEM((GK,), jnp.int32),
            pltpu.VMEM((GK, 128), _F32),
            pltpu.VMEM_SHARED((NN, 128), _F32),
        ],
    )
    def run(t_hbm, ri_hbm, z_hbm, out_hbm, idx_v, v_v, acc):
        c = lax.axis_index("c")
        s = lax.axis_index("s")
        w = s * NC + c
        r0 = s * RPT

        @pl.when(s < NS - 1)
        def _():
            pltpu.sync_copy(z_hbm.at[pl.ds(0, RPT)], acc.at[pl.ds(r0, RPT)])

        @pl.when(s == NS - 1)
        def _():
            pltpu.sync_copy(z_hbm, acc.at[pl.ds(r0, RLAST)])

        plsc.subcore_barrier()
        nt = (SB - w + NW - 1) // NW

        def step(t, carry):
            off = (w + t * NW) * GK
            pltpu.sync_copy(ri_hbm.at[pl.ds(off, GK)], idx_v)
            pltpu.sync_copy(t_hbm.at[pl.ds(off, GK)], v_v)
            pltpu.sync_copy(v_v, acc.at[idx_v], add=True)
            return carry

        lax.fori_loop(0, nt, step, 0)
        plsc.subcore_barrier()

        @pl.when(s < NS - 1)
        def _():
            pltpu.sync_copy(acc.at[pl.ds(r0, RPT)],
                            out_hbm.at[c, pl.ds(r0, RPT)])

        @pl.when(s == NS - 1)
        def _():
            pltpu.sync_copy(acc.at[pl.ds(r0, RLAST)],
                            out_hbm.at[c, pl.ds(r0, RLAST)])

    return run


def _sc_scatter_add16(t16, ri, z):
    return _scatter_add16_kernel()(t16, ri, z)


def _row_spec(b, cdim):
    return pl.BlockSpec((b, cdim), lambda i: (i, 0))


def _full_spec(r, cdim):
    return pl.BlockSpec((r, cdim), lambda i: (0, 0))


def _silu(v):
    return v * jax.nn.sigmoid(v)


def _pack(v):
    lo = jax.lax.bitcast_convert_type(v[:, :128].astype(_BF16), jnp.uint16)
    hi = jax.lax.bitcast_convert_type(v[:, 128:].astype(_BF16), jnp.uint16)
    word = hi.astype(jnp.uint32) << 16 | lo.astype(jnp.uint32)
    return jax.lax.bitcast_convert_type(word, jnp.int32)


def _unpack(w):
    word = jax.lax.bitcast_convert_type(w, jnp.uint32)
    lo = jax.lax.bitcast_convert_type((word & 0xFFFF).astype(jnp.uint16), _BF16)
    hi = jax.lax.bitcast_convert_type((word >> 16).astype(jnp.uint16), _BF16)
    return lo.astype(_F32), hi.astype(_F32)


def _dual_lin_body(h_ref, wa_ref, ba_ref, wb_ref, oa_ref, ob_ref):
    hb = h_ref[...]
    va = _mm(hb, wa_ref[...]) + ba_ref[...]
    vb = _mm(hb, wb_ref[...])
    oa_ref[...] = _pack(va)
    ob_ref[...] = _pack(vb)


def _tc_dual_linear(h, wa, ba, wb):
    return pl.pallas_call(
        _dual_lin_body,
        grid=(NN // BN,),
        in_specs=[_row_spec(BN, H), _full_spec(H, H), _full_spec(1, H),
                  _full_spec(H, H)],
        out_specs=[_row_spec(BN, 128), _row_spec(BN, 128)],
        out_shape=[jax.ShapeDtypeStruct((NN, 128), jnp.int32),
                   jax.ShapeDtypeStruct((NN, 128), jnp.int32)],
    )(h, wa, ba, wb)


def _edge_mlp_body(ga_ref, gb_ref, d_ref, d0_ref, wc_ref, w2_ref, b2_ref,
                   o_ref):
    ga0, ga1 = _unpack(ga_ref[...])
    gb0, gb1 = _unpack(gb_ref[...])
    g0 = ga0 + gb0
    g1 = ga1 + gb1
    d = d_ref[...]
    d0 = d0_ref[...]
    rad = jnp.sum(d * d, axis=1, keepdims=True)
    r0 = jnp.sum(d0 * d0, axis=1, keepdims=True)
    wcb = _rb(wc_ref[...])
    radb = _rb(rad)
    r0b = _rb(r0)
    t0 = _silu(g0 + radb * wcb[0:1, :128] + r0b * wcb[1:2, :128])
    t1 = _silu(g1 + radb * wcb[0:1, 128:] + r0b * wcb[1:2, 128:])
    m = (_mm(t0, w2_ref[0:128, :]) + _mm(t1, w2_ref[128:, :])
         + b2_ref[...])
    o_ref[...] = _silu(m)


def _tc_edge_mlp(ga, gb, d, d0, wc, w2, b2, chunk=0):
    ne = ga.shape[0]
    off = chunk * (ne // BE)
    dspec = pl.BlockSpec((BE, 16), lambda i: (i + off, 0))
    return pl.pallas_call(
        _edge_mlp_body,
        grid=(ne // BE,),
        in_specs=[_row_spec(BE, 128), _row_spec(BE, 128),
                  dspec, dspec,
                  _full_spec(8, H), _full_spec(H, H), _full_spec(1, H)],
        out_specs=_row_spec(BE, H),
        out_shape=jax.ShapeDtypeStruct((ne, H), _F32),
    )(ga, gb, d, d0, wc, w2, b2)


def _node_body(h_ref, a_ref, a2_ref, w1a_ref, w1b_ref, b1_ref, w2_ref,
               b2_ref, o_ref):
    hb = h_ref[...]
    ab = a_ref[...] + a2_ref[...]
    t = (_mm(hb, w1a_ref[...])
         + _mm(ab, w1b_ref[...])
         + b1_ref[...])
    t = _silu(t)
    o_ref[...] = hb + _mm(t, w2_ref[...]) + b2_ref[...]


def _tc_node_mlp(h, agg, agg2, w1a, w1b, b1, w2, b2):
    return pl.pallas_call(
        _node_body,
        grid=(NN // BN,),
        in_specs=[_row_spec(BN, H), _row_spec(BN, H), _row_spec(BN, H),
                  _full_spec(H, H),
                  _full_spec(H, H), _full_spec(1, H), _full_spec(H, H),
                  _full_spec(1, H)],
        out_specs=_row_spec(BN, H),
        out_shape=jax.ShapeDtypeStruct((NN, H), _F32),
    )(h, agg, agg2, w1a, w1b, b1, w2, b2)


def _coord_body(ga_ref, gb_ref, d_ref, d0_ref, wc_ref, w2_ref, b2_ref,
                w3_ref, o_ref):
    ga0, ga1 = _unpack(ga_ref[...])
    gb0, gb1 = _unpack(gb_ref[...])
    g0 = ga0 + gb0
    g1 = ga1 + gb1
    d = d_ref[...]
    d0 = d0_ref[...]
    rad = jnp.sum(d * d, axis=1, keepdims=True)
    r0 = jnp.sum(d0 * d0, axis=1, keepdims=True)
    wcb = _rb(wc_ref[...])
    radb = _rb(rad)
    r0b = _rb(r0)
    t0 = _silu(g0 + radb * wcb[0:1, :128] + r0b * wcb[1:2, :128])
    t1 = _silu(g1 + radb * wcb[0:1, 128:] + r0b * wcb[1:2, 128:])
    m = _silu(_mm(t0, w2_ref[0:128, :]) + _mm(t1, w2_ref[128:, :])
              + b2_ref[...])
    sval = jnp.sum(_rb(m) * _rb(w3_ref[0:1, :]), axis=1, keepdims=True)
    inv = 1.0 / jnp.sqrt(rad + 1e-8)
    t16 = d * (sval * inv)
    o_ref[...] = jnp.concatenate(
        [t16, jnp.zeros((t16.shape[0], 112), _F32)], axis=1)


def _tc_coord_edge(ga, gb, d, d0, wc, w2, b2, w3):
    return pl.pallas_call(
        _coord_body,
        grid=(EE // BE,),
        in_specs=[_row_spec(BE, 128), _row_spec(BE, 128),
                  _row_spec(BE, 16), _row_spec(BE, 16),
                  _full_spec(8, H), _full_spec(H, H), _full_spec(1, H),
                  _full_spec(8, H)],
        out_specs=_row_spec(BE, 128),
        out_shape=jax.ShapeDtypeStruct((EE, 128), _F32),
    )(ga, gb, d, d0, wc, w2, b2, w3)


def _xupd_body(x_ref, a_ref, o_ref):
    o_ref[...] = x_ref[...] + a_ref[0, :, :16] + a_ref[1, :, :16]


def _tc_x_update(x16, aggt):
    return pl.pallas_call(
        _xupd_body,
        grid=(NN // BN,),
        in_specs=[_row_spec(BN, 16),
                  pl.BlockSpec((NC, BN, 128), lambda i: (0, i, 0))],
        out_specs=_row_spec(BN, 16),
        out_shape=jax.ShapeDtypeStruct((NN, 16), _F32),
    )(x16, aggt)


def _emb_body(h_ref, w_ref, b_ref, o_ref):
    o_ref[...] = _mm(h_ref[...], w_ref[...]) + b_ref[...]


def _tc_linear(h, w, b):
    return pl.pallas_call(
        _emb_body,
        grid=(NN // BN,),
        in_specs=[_row_spec(BN, H), _full_spec(H, H), _full_spec(1, H)],
        out_specs=_row_spec(BN, H),
        out_shape=jax.ShapeDtypeStruct((NN, H), _F32),
    )(h, w, b)


def _head_body(h_ref, wo_ref, bo_ref, w1_ref, b1_ref, w2_ref, b2_ref, o_ref):
    hb = h_ref[...]
    o = _mm(hb, wo_ref[...]) + bo_ref[...]
    f = _silu(_mm(o, w1_ref[...]) + b1_ref[...])
    o_ref[...] = _mm(f, w2_ref[...]) + b2_ref[...]


def _tc_head(h, wo, bo, w1, b1, w2p, b2p):
    return pl.pallas_call(
        _head_body,
        grid=(NN // BN,),
        in_specs=[_row_spec(BN, H), _full_spec(H, H), _full_spec(1, H),
                  _full_spec(H, H), _full_spec(1, H), _full_spec(H, OPAD),
                  _full_spec(1, OPAD)],
        out_specs=_row_spec(BN, OPAD),
        out_shape=jax.ShapeDtypeStruct((NN, OPAD), _F32),
    )(h, wo, bo, w1, b1, w2p, b2p)


def _pad_wc(wc2):
    return jnp.zeros((8, H), _F32).at[:2].set(wc2)


def kernel(h, x, edge_index, params):
    row = edge_index[0].astype(jnp.int32)
    col = edge_index[1].astype(jnp.int32)
    x16 = jnp.zeros((NN, 16), _F32).at[:, :3].set(x)

    def _xpad(x16v):
        return jnp.zeros((NN, 128), _F32).at[:, :16].set(x16v)

    z128 = jnp.zeros((RLAST, H // NC), _F32)

    h = _tc_linear(h, params['emb_W'], params['emb_b'].reshape(1, H))

    ech = EE // 2
    rows = (row[:ech], row[ech:])
    cols = (col[:ech], col[ech:])

    d0 = _sc_gather_diff(_xpad(x16), row, col)
    d = d0
    first = True
    for bp in params['blocks']:
        if not first:
            d = _sc_gather_diff(_xpad(x16), row, col)
        first = False
        for gp in bp['gcls']:
            wa = gp['e1_W'][:H]
            wb = gp['e1_W'][H:2 * H]
            wc = _pad_wc(gp['e1_W'][2 * H:])
            ha, hb = _tc_dual_linear(h, wa, gp['e1_b'].reshape(1, H), wb)
            aggs = []
            gs = [_sc_gather_add(ha, hb, rows[k], cols[k]) for k in (0, 1)]
            for k in (0, 1):
                mij = _tc_edge_mlp(gs[k][0], gs[k][1], d, d0, wc,
                                   gp['e2_W'], gp['e2_b'].reshape(1, H),
                                   chunk=k)
                aggs.append(_sc_scatter_add(mij, rows[k], z128))
            h = _tc_node_mlp(h, aggs[0], aggs[1],
                             gp['n1_W'][:H], gp['n1_W'][H:],
                             gp['n1_b'].reshape(1, H),
                             gp['n2_W'], gp['n2_b'].reshape(1, H))
        cp = bp['coord']
        wa = cp['c1_W'][:H]
        wb = cp['c1_W'][H:2 * H]
        wc = _pad_wc(cp['c1_W'][2 * H:])
        ha, hb = _tc_dual_linear(h, wa, cp['c1_b'].reshape(1, H), wb)
        ga, gb = _sc_gather_add(ha, hb, row, col)
        w3 = jnp.zeros((8, H), _F32).at[0].set(cp['c3_W'][:, 0])
        t16 = _tc_coord_edge(ga, gb, d, d0, wc, cp['c2_W'],
                             cp['c2_b'].reshape(1, H), w3)
        aggt = _sc_scatter_add16(t16, row, z128)
        x16 = _tc_x_update(x16, aggt)

    w2p = jnp.zeros((H, OPAD), _F32).at[:, :ODIM].set(params['f2_W'])
    b2p = jnp.zeros((1, OPAD), _F32).at[0, :ODIM].set(params['f2_b'])
    y = _tc_head(h, params['out_W'], params['out_b'].reshape(1, H),
                 params['f1_W'], params['f1_b'].reshape(1, H), w2p, b2p)
    return y[:, :ODIM]


# speedup vs baseline: 2.6756x; 1.0368x over previous
import functools

import jax
import jax.numpy as jnp
from jax import lax
from jax.experimental import pallas as pl
from jax.experimental.pallas import tpu as pltpu
from jax.experimental.pallas import tpu_sc as plsc

NN = 10000
EE = 160000
H = 256
LAT = 64
ODIM = LAT * 2 + 1
OPAD = 256

NC, NS = 2, 16
NW = NC * NS
GK = 128
SB = EE // GK
RPT = 624
RLAST = NN - RPT * (NS - 1)

BN = 1000
BE = 1000

_F32 = jnp.float32
_BF16 = jnp.bfloat16


def _mm(a, w):
    return jnp.dot(a.astype(_BF16), w.astype(_BF16), preferred_element_type=_F32)


def _rb(v):
    return v.astype(_BF16).astype(_F32)


def _mesh():
    return plsc.VectorSubcoreMesh(
        core_axis_name="c", subcore_axis_name="s",
        num_cores=NC, num_subcores=NS)


@functools.cache
def _gather_add_kernel(ne):
    sb = ne // GK

    @functools.partial(
        pl.kernel,
        out_type=(jax.ShapeDtypeStruct((ne, 128), jnp.int32),
                  jax.ShapeDtypeStruct((ne, 128), jnp.int32)),
        mesh=_mesh(),
        scratch_types=[
            pltpu.VMEM((GK,), jnp.int32),
            pltpu.VMEM((GK,), jnp.int32),
            pltpu.VMEM((GK, 128), jnp.int32),
            pltpu.VMEM((GK, 128), jnp.int32),
            pltpu.SemaphoreType.DMA,
        ],
    )
    def run(a_hbm, b_hbm, ri_hbm, ci_hbm, oa_hbm, ob_hbm,
            ia_v, ib_v, a_v, b_v, sem):
        c = lax.axis_index("c")
        s = lax.axis_index("s")
        w = s * NC + c
        nt = (sb - w + NW - 1) // NW

        def step(t, carry):
            off = (w + t * NW) * GK
            pltpu.sync_copy(ri_hbm.at[pl.ds(off, GK)], ia_v)
            pltpu.sync_copy(ci_hbm.at[pl.ds(off, GK)], ib_v)
            ca = pltpu.async_copy(a_hbm.at[ia_v], a_v, sem)
            cb = pltpu.async_copy(b_hbm.at[ib_v], b_v, sem)
            ca.wait()
            cb.wait()
            pltpu.sync_copy(a_v, oa_hbm.at[pl.ds(off, GK)])
            pltpu.sync_copy(b_v, ob_hbm.at[pl.ds(off, GK)])
            return carry

        lax.fori_loop(0, nt, step, 0)

    return run


def _sc_gather_add(a, b, ri, ci):
    return _gather_add_kernel(ri.shape[0])(a, b, ri, ci)


@functools.cache
def _gather_diff_kernel():
    @functools.partial(
        pl.kernel,
        out_type=jax.ShapeDtypeStruct((EE, 16), _F32),
        mesh=_mesh(),
        scratch_types=[
            pltpu.VMEM((GK,), jnp.int32),
            pltpu.VMEM((GK,), jnp.int32),
            pltpu.VMEM((GK, 128), _F32),
            pltpu.VMEM((GK, 128), _F32),
            pltpu.VMEM((GK, 16), _F32),
            pltpu.SemaphoreType.DMA,
        ],
    )
    def run(x_hbm, ri_hbm, ci_hbm, out_hbm, ia_v, ib_v, a_v, b_v, o_v, sem):
        c = lax.axis_index("c")
        s = lax.axis_index("s")
        w = s * NC + c
        nt = (SB - w + NW - 1) // NW

        def step(t, carry):
            off = (w + t * NW) * GK
            pltpu.sync_copy(ri_hbm.at[pl.ds(off, GK)], ia_v)
            pltpu.sync_copy(ci_hbm.at[pl.ds(off, GK)], ib_v)
            ca = pltpu.async_copy(x_hbm.at[ia_v], a_v, sem)
            cb = pltpu.async_copy(x_hbm.at[ib_v], b_v, sem)
            ca.wait()
            cb.wait()

            def sub_row(r, carry2):
                sl = pl.ds(0, 16)
                o_v[r, sl] = a_v[r, sl] - b_v[r, sl]
                return carry2

            lax.fori_loop(0, GK, sub_row, 0)
            pltpu.sync_copy(o_v, out_hbm.at[pl.ds(off, GK)])
            return carry

        lax.fori_loop(0, nt, step, 0)

    return run


def _sc_gather_diff(x16, ri, ci):
    return _gather_diff_kernel()(x16, ri, ci)


@functools.cache
def _scatter_add_kernel(ne):
    hw = H // NC
    sb = ne // GK

    @functools.partial(
        pl.kernel,
        out_type=jax.ShapeDtypeStruct((NN, H), _F32),
        mesh=_mesh(),
        scratch_types=[
            pltpu.VMEM((GK,), jnp.int32),
            pltpu.VMEM((GK, hw), _F32),
            pltpu.VMEM_SHARED((NN, hw), _F32),
        ],
    )
    def run(m_hbm, ri_hbm, z_hbm, out_hbm, idx_v, v_v, acc):
        c = lax.axis_index("c")
        s = lax.axis_index("s")
        r0 = s * RPT

        @pl.when(s < NS - 1)
        def _():
            pltpu.sync_copy(z_hbm.at[pl.ds(0, RPT)], acc.at[pl.ds(r0, RPT)])

        @pl.when(s == NS - 1)
        def _():
            pltpu.sync_copy(z_hbm, acc.at[pl.ds(r0, RLAST)])

        plsc.subcore_barrier()
        nt = (sb - s + NS - 1) // NS

        def step(t, carry):
            off = (s + t * NS) * GK
            pltpu.sync_copy(ri_hbm.at[pl.ds(off, GK)], idx_v)
            pltpu.sync_copy(m_hbm.at[pl.ds(off, GK), pl.ds(c * hw, hw)], v_v)
            pltpu.sync_copy(v_v, acc.at[idx_v], add=True)
            return carry

        lax.fori_loop(0, nt, step, 0)
        plsc.subcore_barrier()

        @pl.when(s < NS - 1)
        def _():
            pltpu.sync_copy(acc.at[pl.ds(r0, RPT)],
                            out_hbm.at[pl.ds(r0, RPT), pl.ds(c * hw, hw)])

        @pl.when(s == NS - 1)
        def _():
            pltpu.sync_copy(acc.at[pl.ds(r0, RLAST)],
                            out_hbm.at[pl.ds(r0, RLAST), pl.ds(c * hw, hw)])

    return run


def _sc_scatter_add(m, ri, z):
    return _scatter_add_kernel(ri.shape[0])(m, ri, z)


@functools.cache
def _scatter_add16_kernel(ne):
    sb = ne // GK

    @functools.partial(
        pl.kernel,
        out_type=jax.ShapeDtypeStruct((NC, NN, 128), _F32),
        mesh=_mesh(),
        scratch_types=[
            pltpu.VMEM((GK,), jnp.int32),
            pltpu.VMEM((GK, 128), _F32),
            pltpu.VMEM_SHARED((NN, 128), _F32),
        ],
    )
    def run(t_hbm, ri_hbm, z_hbm, out_hbm, idx_v, v_v, acc):
        c = lax.axis_index("c")
        s = lax.axis_index("s")
        w = s * NC + c
        r0 = s * RPT

        @pl.when(s < NS - 1)
        def _():
            pltpu.sync_copy(z_hbm.at[pl.ds(0, RPT)], acc.at[pl.ds(r0, RPT)])

        @pl.when(s == NS - 1)
        def _():
            pltpu.sync_copy(z_hbm, acc.at[pl.ds(r0, RLAST)])

        plsc.subcore_barrier()
        nt = (sb - w + NW - 1) // NW

        def step(t, carry):
            off = (w + t * NW) * GK
            pltpu.sync_copy(ri_hbm.at[pl.ds(off, GK)], idx_v)
            pltpu.sync_copy(t_hbm.at[pl.ds(off, GK)], v_v)
            pltpu.sync_copy(v_v, acc.at[idx_v], add=True)
            return carry

        lax.fori_loop(0, nt, step, 0)
        plsc.subcore_barrier()

        @pl.when(s < NS - 1)
        def _():
            pltpu.sync_copy(acc.at[pl.ds(r0, RPT)],
                            out_hbm.at[c, pl.ds(r0, RPT)])

        @pl.when(s == NS - 1)
        def _():
            pltpu.sync_copy(acc.at[pl.ds(r0, RLAST)],
                            out_hbm.at[c, pl.ds(r0, RLAST)])

    return run


def _sc_scatter_add16(t16, ri, z):
    return _scatter_add16_kernel(ri.shape[0])(t16, ri, z)


def _row_spec(b, cdim):
    return pl.BlockSpec((b, cdim), lambda i: (i, 0))


def _full_spec(r, cdim):
    return pl.BlockSpec((r, cdim), lambda i: (0, 0))


def _silu(v):
    return v * jax.nn.sigmoid(v)


def _pack(v):
    lo = jax.lax.bitcast_convert_type(v[:, :128].astype(_BF16), jnp.uint16)
    hi = jax.lax.bitcast_convert_type(v[:, 128:].astype(_BF16), jnp.uint16)
    word = hi.astype(jnp.uint32) << 16 | lo.astype(jnp.uint32)
    return jax.lax.bitcast_convert_type(word, jnp.int32)


def _unpack(w):
    word = jax.lax.bitcast_convert_type(w, jnp.uint32)
    lo = jax.lax.bitcast_convert_type((word & 0xFFFF).astype(jnp.uint16), _BF16)
    hi = jax.lax.bitcast_convert_type((word >> 16).astype(jnp.uint16), _BF16)
    return lo.astype(_F32), hi.astype(_F32)


def _dual_lin_body(h_ref, wa_ref, ba_ref, wb_ref, oa_ref, ob_ref):
    hb = h_ref[...]
    va = _mm(hb, wa_ref[...]) + ba_ref[...]
    vb = _mm(hb, wb_ref[...])
    oa_ref[...] = _pack(va)
    ob_ref[...] = _pack(vb)


def _tc_dual_linear(h, wa, ba, wb):
    return pl.pallas_call(
        _dual_lin_body,
        grid=(NN // BN,),
        in_specs=[_row_spec(BN, H), _full_spec(H, H), _full_spec(1, H),
                  _full_spec(H, H)],
        out_specs=[_row_spec(BN, 128), _row_spec(BN, 128)],
        out_shape=[jax.ShapeDtypeStruct((NN, 128), jnp.int32),
                   jax.ShapeDtypeStruct((NN, 128), jnp.int32)],
    )(h, wa, ba, wb)


def _edge_mlp_body(ga_ref, gb_ref, d_ref, d0_ref, wc_ref, w2_ref, b2_ref,
                   o_ref):
    ga0, ga1 = _unpack(ga_ref[...])
    gb0, gb1 = _unpack(gb_ref[...])
    g0 = ga0 + gb0
    g1 = ga1 + gb1
    d = d_ref[...]
    d0 = d0_ref[...]
    rad = jnp.sum(d * d, axis=1, keepdims=True)
    r0 = jnp.sum(d0 * d0, axis=1, keepdims=True)
    wcb = _rb(wc_ref[...])
    radb = _rb(rad)
    r0b = _rb(r0)
    t0 = _silu(g0 + radb * wcb[0:1, :128] + r0b * wcb[1:2, :128])
    t1 = _silu(g1 + radb * wcb[0:1, 128:] + r0b * wcb[1:2, 128:])
    m = (_mm(t0, w2_ref[0:128, :]) + _mm(t1, w2_ref[128:, :])
         + b2_ref[...])
    o_ref[...] = _silu(m)


def _tc_edge_mlp(ga, gb, d, d0, wc, w2, b2, chunk=0):
    ne = ga.shape[0]
    off = chunk * (ne // BE)
    dspec = pl.BlockSpec((BE, 16), lambda i: (i + off, 0))
    return pl.pallas_call(
        _edge_mlp_body,
        grid=(ne // BE,),
        in_specs=[_row_spec(BE, 128), _row_spec(BE, 128),
                  dspec, dspec,
                  _full_spec(8, H), _full_spec(H, H), _full_spec(1, H)],
        out_specs=_row_spec(BE, H),
        out_shape=jax.ShapeDtypeStruct((ne, H), _F32),
    )(ga, gb, d, d0, wc, w2, b2)


def _node_body(h_ref, a_ref, a2_ref, w1a_ref, w1b_ref, b1_ref, w2_ref,
               b2_ref, o_ref):
    hb = h_ref[...]
    ab = a_ref[...] + a2_ref[...]
    t = (_mm(hb, w1a_ref[...])
         + _mm(ab, w1b_ref[...])
         + b1_ref[...])
    t = _silu(t)
    o_ref[...] = hb + _mm(t, w2_ref[...]) + b2_ref[...]


def _tc_node_mlp(h, agg, agg2, w1a, w1b, b1, w2, b2):
    return pl.pallas_call(
        _node_body,
        grid=(NN // BN,),
        in_specs=[_row_spec(BN, H), _row_spec(BN, H), _row_spec(BN, H),
                  _full_spec(H, H),
                  _full_spec(H, H), _full_spec(1, H), _full_spec(H, H),
                  _full_spec(1, H)],
        out_specs=_row_spec(BN, H),
        out_shape=jax.ShapeDtypeStruct((NN, H), _F32),
    )(h, agg, agg2, w1a, w1b, b1, w2, b2)


def _coord_body(ga_ref, gb_ref, d_ref, d0_ref, wc_ref, w2_ref, b2_ref,
                w3_ref, o_ref):
    ga0, ga1 = _unpack(ga_ref[...])
    gb0, gb1 = _unpack(gb_ref[...])
    g0 = ga0 + gb0
    g1 = ga1 + gb1
    d = d_ref[...]
    d0 = d0_ref[...]
    rad = jnp.sum(d * d, axis=1, keepdims=True)
    r0 = jnp.sum(d0 * d0, axis=1, keepdims=True)
    wcb = _rb(wc_ref[...])
    radb = _rb(rad)
    r0b = _rb(r0)
    t0 = _silu(g0 + radb * wcb[0:1, :128] + r0b * wcb[1:2, :128])
    t1 = _silu(g1 + radb * wcb[0:1, 128:] + r0b * wcb[1:2, 128:])
    m = _silu(_mm(t0, w2_ref[0:128, :]) + _mm(t1, w2_ref[128:, :])
              + b2_ref[...])
    sval = jnp.sum(_rb(m) * _rb(w3_ref[0:1, :]), axis=1, keepdims=True)
    inv = 1.0 / jnp.sqrt(rad + 1e-8)
    t16 = d * (sval * inv)
    o_ref[...] = jnp.concatenate(
        [t16, jnp.zeros((t16.shape[0], 112), _F32)], axis=1)


def _tc_coord_edge(ga, gb, d, d0, wc, w2, b2, w3, chunk=0):
    ne = ga.shape[0]
    off = chunk * (ne // BE)
    dspec = pl.BlockSpec((BE, 16), lambda i: (i + off, 0))
    return pl.pallas_call(
        _coord_body,
        grid=(ne // BE,),
        in_specs=[_row_spec(BE, 128), _row_spec(BE, 128),
                  dspec, dspec,
                  _full_spec(8, H), _full_spec(H, H), _full_spec(1, H),
                  _full_spec(8, H)],
        out_specs=_row_spec(BE, 128),
        out_shape=jax.ShapeDtypeStruct((ne, 128), _F32),
    )(ga, gb, d, d0, wc, w2, b2, w3)


def _xupd_body(x_ref, a_ref, b_ref, o_ref):
    o_ref[...] = (x_ref[...] + a_ref[0, :, :16] + a_ref[1, :, :16]
                  + b_ref[0, :, :16] + b_ref[1, :, :16])


def _tc_x_update(x16, aggt, aggt2):
    aspec = pl.BlockSpec((NC, BN, 128), lambda i: (0, i, 0))
    return pl.pallas_call(
        _xupd_body,
        grid=(NN // BN,),
        in_specs=[_row_spec(BN, 16), aspec, aspec],
        out_specs=_row_spec(BN, 16),
        out_shape=jax.ShapeDtypeStruct((NN, 16), _F32),
    )(x16, aggt, aggt2)


def _emb_body(h_ref, w_ref, b_ref, o_ref):
    o_ref[...] = _mm(h_ref[...], w_ref[...]) + b_ref[...]


def _tc_linear(h, w, b):
    return pl.pallas_call(
        _emb_body,
        grid=(NN // BN,),
        in_specs=[_row_spec(BN, H), _full_spec(H, H), _full_spec(1, H)],
        out_specs=_row_spec(BN, H),
        out_shape=jax.ShapeDtypeStruct((NN, H), _F32),
    )(h, w, b)


def _head_body(h_ref, wo_ref, bo_ref, w1_ref, b1_ref, w2_ref, b2_ref, o_ref):
    hb = h_ref[...]
    o = _mm(hb, wo_ref[...]) + bo_ref[...]
    f = _silu(_mm(o, w1_ref[...]) + b1_ref[...])
    o_ref[...] = _mm(f, w2_ref[...]) + b2_ref[...]


def _tc_head(h, wo, bo, w1, b1, w2p, b2p):
    return pl.pallas_call(
        _head_body,
        grid=(NN // BN,),
        in_specs=[_row_spec(BN, H), _full_spec(H, H), _full_spec(1, H),
                  _full_spec(H, H), _full_spec(1, H), _full_spec(H, OPAD),
                  _full_spec(1, OPAD)],
        out_specs=_row_spec(BN, OPAD),
        out_shape=jax.ShapeDtypeStruct((NN, OPAD), _F32),
    )(h, wo, bo, w1, b1, w2p, b2p)


def _pad_wc(wc2):
    return jnp.zeros((8, H), _F32).at[:2].set(wc2)


def kernel(h, x, edge_index, params):
    row = edge_index[0].astype(jnp.int32)
    col = edge_index[1].astype(jnp.int32)
    x16 = jnp.zeros((NN, 16), _F32).at[:, :3].set(x)

    def _xpad(x16v):
        return jnp.zeros((NN, 128), _F32).at[:, :16].set(x16v)

    z128 = jnp.zeros((RLAST, H // NC), _F32)

    h = _tc_linear(h, params['emb_W'], params['emb_b'].reshape(1, H))

    ech = EE // 2
    rows = (row[:ech], row[ech:])
    cols = (col[:ech], col[ech:])

    d0 = _sc_gather_diff(_xpad(x16), row, col)
    d = d0
    first = True
    for bp in params['blocks']:
        if not first:
            d = _sc_gather_diff(_xpad(x16), row, col)
        first = False
        for gp in bp['gcls']:
            wa = gp['e1_W'][:H]
            wb = gp['e1_W'][H:2 * H]
            wc = _pad_wc(gp['e1_W'][2 * H:])
            ha, hb = _tc_dual_linear(h, wa, gp['e1_b'].reshape(1, H), wb)
            aggs = []
            gs = [_sc_gather_add(ha, hb, rows[k], cols[k]) for k in (0, 1)]
            for k in (0, 1):
                mij = _tc_edge_mlp(gs[k][0], gs[k][1], d, d0, wc,
                                   gp['e2_W'], gp['e2_b'].reshape(1, H),
                                   chunk=k)
                aggs.append(_sc_scatter_add(mij, rows[k], z128))
            h = _tc_node_mlp(h, aggs[0], aggs[1],
                             gp['n1_W'][:H], gp['n1_W'][H:],
                             gp['n1_b'].reshape(1, H),
                             gp['n2_W'], gp['n2_b'].reshape(1, H))
        cp = bp['coord']
        wa = cp['c1_W'][:H]
        wb = cp['c1_W'][H:2 * H]
        wc = _pad_wc(cp['c1_W'][2 * H:])
        ha, hb = _tc_dual_linear(h, wa, cp['c1_b'].reshape(1, H), wb)
        w3 = jnp.zeros((8, H), _F32).at[0].set(cp['c3_W'][:, 0])
        gsc = [_sc_gather_add(ha, hb, rows[k], cols[k]) for k in (0, 1)]
        aggts = []
        for k in (0, 1):
            t16 = _tc_coord_edge(gsc[k][0], gsc[k][1], d, d0, wc,
                                 cp['c2_W'], cp['c2_b'].reshape(1, H), w3,
                                 chunk=k)
            aggts.append(_sc_scatter_add16(t16, rows[k], z128))
        x16 = _tc_x_update(x16, aggts[0], aggts[1])

    w2p = jnp.zeros((H, OPAD), _F32).at[:, :ODIM].set(params['f2_W'])
    b2p = jnp.zeros((1, OPAD), _F32).at[0, :ODIM].set(params['f2_b'])
    y = _tc_head(h, params['out_W'], params['out_b'].reshape(1, H),
                 params['f1_W'], params['f1_b'].reshape(1, H), w2p, b2p)
    return y[:, :ODIM]
